# Initial kernel scaffold; baseline (speedup 1.0000x reference)
import jax
import jax.numpy as jnp
from jax.experimental import pallas as pl


def kernel(preds, targets):
    raise NotImplementedError("write your pallas kernel here")


# speedup vs baseline: 25.9541x; 25.9541x over previous
import functools

import jax
import jax.numpy as jnp
from jax.experimental import pallas as pl
from jax.experimental.pallas import tpu as pltpu

_BINS = 30
_ROWS_PER_BLOCK = 64


def _ghm_kernel(p_ref, t_ref, out_ref, acc_ref, *, n_classes, n_steps):
    i = pl.program_id(0)

    @pl.when(i == 0)
    def _init():
        acc_ref[...] = jnp.zeros_like(acc_ref)

    p0 = p_ref[0, 0]
    t = t_ref[0]

    m = p0
    for c in range(1, n_classes):
        m = jnp.maximum(m, p_ref[0, c])

    denom = jnp.exp(p0 - m)
    pt = jnp.where(t == 0, p0, 0.0)
    for c in range(1, n_classes):
        pc = p_ref[0, c]
        denom = denom + jnp.exp(pc - m)
        pt = pt + jnp.where(t == c, pc, 0.0)

    shifted_t = pt - m
    ce = jnp.log(denom) - shifted_t
    py = jnp.exp(shifted_t) / denom
    g = jnp.abs(py - 1.0)

    kc = jnp.clip(jnp.floor(g * 30.0).astype(jnp.int32), 0, _BINS - 1)
    kcf = kc.astype(jnp.float32)
    e_lo = kcf / 30.0
    e_hi = (kcf + 1.0) / 30.0
    wb = kc - (g <= e_lo).astype(jnp.int32) + (g > e_hi).astype(jnp.int32)
    r30 = jnp.float32(1.0) / jnp.float32(30.0)
    eh_lo = kcf * r30
    eh_hi = (kcf + 1.0) * r30
    hb = kc - (g < eh_lo).astype(jnp.int32) + (g >= eh_hi).astype(jnp.int32)
    hb = jnp.minimum(hb, _BINS - 1)

    for b in range(_BINS):
        mh = (hb == b).astype(jnp.float32)
        mw = wb == b
        mwf = mw.astype(jnp.float32)
        acc_ref[b, :] += jnp.sum(mh, axis=0)
        acc_ref[_BINS + 2 + b, :] += jnp.sum(jnp.where(mw, ce, 0.0), axis=0)
        acc_ref[2 * (_BINS + 2) + b, :] += jnp.sum(mwf, axis=0)

    @pl.when(i == n_steps - 1)
    def _fin():
        acc = acc_ref[...]
        c_b = jnp.sum(acc[0:_BINS + 2, :], axis=1, keepdims=True)
        s_b = jnp.sum(acc[_BINS + 2:2 * (_BINS + 2), :], axis=1, keepdims=True)
        n_b = jnp.sum(acc[2 * (_BINS + 2):3 * (_BINS + 2), :], axis=1, keepdims=True)
        valid = n_b > 0.0
        num = jnp.sum(jnp.where(valid, s_b / c_b, 0.0))
        den = jnp.sum(jnp.where(valid, n_b / c_b, 0.0)) + 1e-7
        out_ref[...] = jnp.full(out_ref.shape, num / den, jnp.float32)


def kernel(preds, targets):
    b, n_classes, h, w = preds.shape
    hb = _ROWS_PER_BLOCK
    steps_per_image = h // hb
    n_steps = b * steps_per_image

    out = pl.pallas_call(
        functools.partial(_ghm_kernel, n_classes=n_classes, n_steps=n_steps),
        grid=(n_steps,),
        in_specs=[
            pl.BlockSpec((1, n_classes, hb, w),
                         lambda i: (i // steps_per_image, 0, i % steps_per_image, 0)),
            pl.BlockSpec((1, hb, w),
                         lambda i: (i // steps_per_image, i % steps_per_image, 0)),
        ],
        out_specs=pl.BlockSpec((8, 128), lambda i: (0, 0)),
        out_shape=jax.ShapeDtypeStruct((8, 128), jnp.float32),
        scratch_shapes=[pltpu.VMEM((3 * (_BINS + 2), w), jnp.float32)],
    )(preds, targets)
    return out[0, 0]
